# Initial kernel scaffold; baseline (speedup 1.0000x reference)
import jax
import jax.numpy as jnp
from jax.experimental import pallas as pl


def kernel(token_ids, table):
    raise NotImplementedError("write your pallas kernel here")


# speedup vs baseline: 1.1076x; 1.1076x over previous
import functools

import jax
import jax.numpy as jnp
from jax import lax
from jax.experimental import pallas as pl
from jax.experimental.pallas import tpu as pltpu
from jax.experimental.pallas import tpu_sc as plsc


def _embedding_gather(flat_ids, table, *, b_per_w, chunk, num_workers, dim):
    n_chunks = b_per_w // chunk
    mesh = plsc.VectorSubcoreMesh(core_axis_name="c", subcore_axis_name="s")
    total = b_per_w * num_workers

    @functools.partial(
        pl.kernel,
        out_type=jax.ShapeDtypeStruct((total, dim), jnp.float32),
        mesh=mesh,
        scratch_types=[
            pltpu.VMEM((b_per_w,), jnp.int32),
            pltpu.VMEM((chunk, dim), jnp.float32),
            pltpu.SemaphoreType.DMA,
        ],
        compiler_params=pltpu.CompilerParams(use_tc_tiling_on_sc=False),
    )
    def k(ids_hbm, table_hbm, out_hbm, idx_v, rows_v, sem):
        wid = lax.axis_index("s") * 2 + lax.axis_index("c")
        base = wid * b_per_w
        pltpu.sync_copy(ids_hbm.at[pl.ds(base, b_per_w)], idx_v)

        def body(i, _):
            off = i * chunk
            pltpu.async_copy(
                table_hbm.at[idx_v.at[pl.ds(off, chunk)]], rows_v, sem
            ).wait()
            pltpu.sync_copy(rows_v, out_hbm.at[pl.ds(base + off, chunk)])
            return 0

        lax.fori_loop(0, n_chunks, body, 0)

    return k(flat_ids, table)


def kernel(token_ids, table):
    b0, b1 = token_ids.shape
    num_rows, dim = table.shape
    total = b0 * b1
    num_workers = 32
    b_per_w = total // num_workers
    chunk = 1600

    flat_ids = token_ids.reshape(total).astype(jnp.int32)
    out = _embedding_gather(
        flat_ids, table,
        b_per_w=b_per_w, chunk=chunk, num_workers=num_workers, dim=dim,
    )
    return out.reshape(b0, b1, dim)


# speedup vs baseline: 1.1134x; 1.0052x over previous
import functools

import jax
import jax.numpy as jnp
from jax import lax
from jax.experimental import pallas as pl
from jax.experimental.pallas import tpu as pltpu
from jax.experimental.pallas import tpu_sc as plsc

_NBUF = 4


def _embedding_gather(flat_ids, table, *, b_per_w, chunk, num_workers, dim):
    n_chunks = b_per_w // chunk
    mesh = plsc.VectorSubcoreMesh(core_axis_name="c", subcore_axis_name="s")
    total = b_per_w * num_workers

    @functools.partial(
        pl.kernel,
        out_type=jax.ShapeDtypeStruct((total, dim), jnp.float32),
        mesh=mesh,
        scratch_types=[
            pltpu.VMEM((b_per_w,), jnp.int32),
            [pltpu.VMEM((chunk, dim), jnp.float32) for _ in range(_NBUF)],
            [pltpu.SemaphoreType.DMA for _ in range(_NBUF)],
            [pltpu.SemaphoreType.DMA for _ in range(_NBUF)],
        ],
        compiler_params=pltpu.CompilerParams(use_tc_tiling_on_sc=False),
    )
    def k(ids_hbm, table_hbm, out_hbm, idx_v, rows, g_sems, w_sems):
        wid = lax.axis_index("s") * 2 + lax.axis_index("c")
        base = wid * b_per_w
        pltpu.sync_copy(ids_hbm.at[pl.ds(base, b_per_w)], idx_v)

        gather_descs = [None] * n_chunks
        write_descs = [None] * n_chunks

        def writeback(j):
            gather_descs[j].wait()
            b = j % _NBUF
            write_descs[j] = pltpu.async_copy(
                rows[b], out_hbm.at[pl.ds(base + j * chunk, chunk)], w_sems[b]
            )

        for i in range(n_chunks):
            b = i % _NBUF
            if i >= _NBUF:
                write_descs[i - _NBUF].wait()
            gather_descs[i] = pltpu.async_copy(
                table_hbm.at[idx_v.at[pl.ds(i * chunk, chunk)]], rows[b], g_sems[b]
            )
            j = i - (_NBUF - 1)
            if j >= 0:
                writeback(j)
        for j in range(max(0, n_chunks - (_NBUF - 1)), n_chunks):
            writeback(j)
        for j in range(max(0, n_chunks - _NBUF), n_chunks):
            write_descs[j].wait()

    return k(flat_ids, table)


def kernel(token_ids, table):
    b0, b1 = token_ids.shape
    num_rows, dim = table.shape
    total = b0 * b1
    num_workers = 32
    b_per_w = total // num_workers
    chunk = 640

    flat_ids = token_ids.reshape(total).astype(jnp.int32)
    out = _embedding_gather(
        flat_ids, table,
        b_per_w=b_per_w, chunk=chunk, num_workers=num_workers, dim=dim,
    )
    return out.reshape(b0, b1, dim)


# speedup vs baseline: 1.9394x; 1.7419x over previous
import functools

import jax
import jax.numpy as jnp
from jax import lax
from jax.experimental import pallas as pl
from jax.experimental.pallas import tpu as pltpu
from jax.experimental.pallas import tpu_sc as plsc

_NBUF = 4


def _embedding_gather(ids_t, table, *, num_workers):
    n_rows, n_tok = ids_t.shape
    dim = table.shape[1]
    chunk = n_tok // num_workers
    mesh = plsc.VectorSubcoreMesh(core_axis_name="c", subcore_axis_name="s")

    @functools.partial(
        pl.kernel,
        out_type=jax.ShapeDtypeStruct((n_rows, n_tok, dim), jnp.float32),
        mesh=mesh,
        scratch_types=[
            pltpu.VMEM((n_rows, chunk), jnp.int32),
            [pltpu.VMEM((chunk, dim), jnp.float32) for _ in range(_NBUF)],
            [pltpu.SemaphoreType.DMA for _ in range(_NBUF)],
            [pltpu.SemaphoreType.DMA for _ in range(_NBUF)],
        ],
        compiler_params=pltpu.CompilerParams(use_tc_tiling_on_sc=False),
    )
    def k(ids_hbm, table_hbm, out_hbm, idx_v, rows, g_sems, w_sems):
        wid = lax.axis_index("s") * 2 + lax.axis_index("c")
        base = wid * chunk
        pltpu.sync_copy(ids_hbm.at[:, pl.ds(base, chunk)], idx_v)

        gather_descs = [None] * n_rows
        write_descs = [None] * n_rows

        def writeback(j):
            gather_descs[j].wait()
            b = j % _NBUF
            write_descs[j] = pltpu.async_copy(
                rows[b], out_hbm.at[j, pl.ds(base, chunk), :], w_sems[b]
            )

        for i in range(n_rows):
            b = i % _NBUF
            if i >= _NBUF:
                write_descs[i - _NBUF].wait()
            gather_descs[i] = pltpu.async_copy(
                table_hbm.at[idx_v.at[i]], rows[b], g_sems[b]
            )
            j = i - (_NBUF - 1)
            if j >= 0:
                writeback(j)
        for j in range(max(0, n_rows - (_NBUF - 1)), n_rows):
            writeback(j)
        for j in range(max(0, n_rows - _NBUF), n_rows):
            write_descs[j].wait()

    return k(ids_t, table)


def kernel(token_ids, table):
    ids_t = token_ids.T.astype(jnp.int32)
    out_t = _embedding_gather(ids_t, table, num_workers=32)
    return out_t.transpose(1, 0, 2)
